# baseline (speedup 1.0000x reference)
import jax
import jax.numpy as jnp
from jax import lax
from jax.experimental import pallas as pl
from jax.experimental.pallas import tpu as pltpu
from jax.experimental.pallas import tpu_sc as plsc

VOCAB = 1000000
EMBED = 64
SEQ = 200
BATCH = 4096

_NC = 2
_NS = 16
_NW = _NC * _NS
_BPW = BATCH // _NW
_CB = 2
_CHUNKS = _BPW // _CB
_ROWS = _CB * SEQ
_SPLIT = (104, 96)


def _pool_body(x_hbm, emb_hbm, out_hbm, slab_v, idxt_v, rows0, rows1,
               stage_v, sem0, sem1):
    wid = lax.axis_index("s") * _NC + lax.axis_index("c")
    base0 = wid * _BPW
    inv = jnp.float32(1.0 / SEQ)
    z = jnp.zeros((16,), jnp.float32)

    pltpu.sync_copy(x_hbm.at[:, pl.ds(base0, _BPW)], slab_v)
    ci = jnp.arange(16, dtype=jnp.int32) * SEQ

    def tr_body(r, carry):
        for e0 in range(0, _BPW, 16):
            v = slab_v[r, pl.ds(e0, 16)]
            plsc.store_scatter(idxt_v, [ci + (e0 * SEQ + r)], v)
        return carry

    lax.fori_loop(0, SEQ, tr_body, 0)

    def fire(g, rows_v, sem):
        for e in range(_CB):
            off = (g * _CB + e) * SEQ
            pltpu.async_copy(
                emb_hbm.at[idxt_v.at[pl.ds(off, _SPLIT[0])]],
                rows_v.at[pl.ds(e * SEQ, _SPLIT[0]), :],
                sem,
            )
            pltpu.async_copy(
                emb_hbm.at[idxt_v.at[pl.ds(off + _SPLIT[0], _SPLIT[1])]],
                rows_v.at[pl.ds(e * SEQ + _SPLIT[0], _SPLIT[1]), :],
                sem,
            )

    def drain(g, rows_v, sem):
        for e in range(_CB):
            off = (g * _CB + e) * SEQ
            pltpu.make_async_copy(
                emb_hbm.at[idxt_v.at[pl.ds(off, _SPLIT[0])]],
                rows_v.at[pl.ds(e * SEQ, _SPLIT[0]), :],
                sem,
            ).wait()
            pltpu.make_async_copy(
                emb_hbm.at[idxt_v.at[pl.ds(off + _SPLIT[0], _SPLIT[1])]],
                rows_v.at[pl.ds(e * SEQ + _SPLIT[0], _SPLIT[1]), :],
                sem,
            ).wait()

    def accum(g, rows_v):
        for e in range(_CB):
            def row_body(r, acc):
                b0, b1, b2, b3, c0, c1, c2, c3 = acc
                r0 = e * SEQ + r * 4
                b0 = b0 + rows_v[r0, pl.ds(0, 16)]
                b1 = b1 + rows_v[r0, pl.ds(16, 16)]
                b2 = b2 + rows_v[r0, pl.ds(32, 16)]
                b3 = b3 + rows_v[r0, pl.ds(48, 16)]
                c0 = c0 + rows_v[r0 + 1, pl.ds(0, 16)]
                c1 = c1 + rows_v[r0 + 1, pl.ds(16, 16)]
                c2 = c2 + rows_v[r0 + 1, pl.ds(32, 16)]
                c3 = c3 + rows_v[r0 + 1, pl.ds(48, 16)]
                b0 = b0 + rows_v[r0 + 2, pl.ds(0, 16)]
                b1 = b1 + rows_v[r0 + 2, pl.ds(16, 16)]
                b2 = b2 + rows_v[r0 + 2, pl.ds(32, 16)]
                b3 = b3 + rows_v[r0 + 2, pl.ds(48, 16)]
                c0 = c0 + rows_v[r0 + 3, pl.ds(0, 16)]
                c1 = c1 + rows_v[r0 + 3, pl.ds(16, 16)]
                c2 = c2 + rows_v[r0 + 3, pl.ds(32, 16)]
                c3 = c3 + rows_v[r0 + 3, pl.ds(48, 16)]
                return (b0, b1, b2, b3, c0, c1, c2, c3)

            b0, b1, b2, b3, c0, c1, c2, c3 = lax.fori_loop(
                0, SEQ // 4, row_body, (z, z, z, z, z, z, z, z))
            stage_v[e, pl.ds(0, 16)] = (b0 + c0) * inv
            stage_v[e, pl.ds(16, 16)] = (b1 + c1) * inv
            stage_v[e, pl.ds(32, 16)] = (b2 + c2) * inv
            stage_v[e, pl.ds(48, 16)] = (b3 + c3) * inv

        pltpu.sync_copy(stage_v, out_hbm.at[pl.ds(base0 + g * _CB, _CB), :])

    fire(0, rows0, sem0)

    def body(i, carry):
        g = 2 * i
        fire(g + 1, rows1, sem1)
        drain(g, rows0, sem0)
        accum(g, rows0)
        fire(g + 2, rows0, sem0)
        drain(g + 1, rows1, sem1)
        accum(g + 1, rows1)
        return carry

    lax.fori_loop(0, _CHUNKS // 2 - 1, body, 0)
    g = _CHUNKS - 2
    fire(g + 1, rows1, sem1)
    drain(g, rows0, sem0)
    accum(g, rows0)
    drain(g + 1, rows1, sem1)
    accum(g + 1, rows1)


def _sc_pool(x, emb_table):
    mesh = plsc.VectorSubcoreMesh(
        core_axis_name="c", subcore_axis_name="s",
        num_cores=_NC, num_subcores=_NS,
    )
    f = pl.kernel(
        _pool_body,
        out_type=jax.ShapeDtypeStruct((BATCH, EMBED), jnp.float32),
        mesh=mesh,
        scratch_types=[
            pltpu.VMEM((SEQ, _BPW), jnp.int32),
            pltpu.VMEM((_BPW * SEQ,), jnp.int32),
            pltpu.VMEM((_ROWS, EMBED), jnp.float32),
            pltpu.VMEM((_ROWS, EMBED), jnp.float32),
            pltpu.VMEM((_CB, EMBED), jnp.float32),
            pltpu.SemaphoreType.DMA,
            pltpu.SemaphoreType.DMA,
        ],
        compiler_params=pltpu.CompilerParams(
            use_tc_tiling_on_sc=False, needs_layout_passes=False),
    )
    return f(x, emb_table)


def _mlp_body(p_ref, w1_ref, b1_ref, w2_ref, b2_ref, out_ref):
    p = p_ref[...]
    h = jnp.dot(p, w1_ref[...], preferred_element_type=jnp.float32) + b1_ref[...]
    z = jnp.dot(h, w2_ref[...], preferred_element_type=jnp.float32) + b2_ref[...]
    m = jnp.max(z, axis=-1, keepdims=True)
    e = jnp.exp(z - m)
    out_ref[...] = e / jnp.sum(e, axis=-1, keepdims=True)


def _tc_mlp(pooled, w1t, b1, w2t, b2):
    return pl.pallas_call(
        _mlp_body,
        out_shape=jax.ShapeDtypeStruct((BATCH, 2), jnp.float32),
    )(pooled, w1t, b1, w2t, b2)


@jax.jit
def kernel(x, emb_table, fc1_w, fc1_b, fc2_w, fc2_b):
    pooled = _sc_pool(x, emb_table)
    return _tc_mlp(
        pooled,
        fc1_w.T,
        fc1_b.reshape(1, 10),
        fc2_w.T,
        fc2_b.reshape(1, 2),
    )
